# Initial kernel scaffold; baseline (speedup 1.0000x reference)
import jax
import jax.numpy as jnp
from jax.experimental import pallas as pl


def kernel(x, w_router, W1, W2):
    raise NotImplementedError("write your pallas kernel here")


# speedup vs baseline: 1889.4670x; 1889.4670x over previous
import functools

import jax
import jax.numpy as jnp
from jax.experimental import pallas as pl


_CAPACITY_RATIO = 0.75


def _select_body(s_ref, m_ref, *, k, l):
    s = s_ref[:]
    bits = jax.lax.bitcast_convert_type(s, jnp.int32)
    key = jnp.where(bits >= 0, bits, bits ^ jnp.int32(0x7FFFFFFF))
    kk = jnp.int32(k)

    def cge(t):
        return jnp.sum((key >= t).astype(jnp.int32), axis=1, keepdims=True)

    zero = jnp.zeros((s.shape[0], 1), jnp.int32)
    neg = jnp.full_like(zero, jnp.int32(-2147483648))
    t = jnp.where(cge(zero) >= kk, zero, neg)
    for bit in range(30, -1, -1):
        cand = t + jnp.int32(1 << bit)
        t = jnp.where(cge(cand) >= kk, cand, t)

    gt = key > t
    cnt_gt = jnp.sum(gt.astype(jnp.int32), axis=1, keepdims=True)
    need = kk - cnt_gt
    tie = key == t
    idx = jax.lax.broadcasted_iota(jnp.int32, s.shape, 1)
    bound = jnp.full_like(zero, jnp.int32(l))
    big = jnp.zeros_like(zero)
    for bit in range(12, -1, -1):
        cand = big + jnp.int32(1 << bit)
        cnt = jnp.sum((tie & (idx < cand)).astype(jnp.int32), axis=1,
                      keepdims=True)
        big = jnp.where((cand <= bound) & (cnt <= need), cand, big)

    sel = gt | (tie & (idx < big))
    m_ref[:] = sel.astype(jnp.float32)


def _mlp_body(m_ref, x_ref, w1_ref, w2_ref, o_ref):
    xb = x_ref[0]
    h = jnp.dot(xb.astype(jnp.bfloat16), w1_ref[:],
                preferred_element_type=jnp.float32)
    h = jnp.maximum(h, 0.0).astype(jnp.bfloat16)
    y = jnp.dot(h, w2_ref[:], preferred_element_type=jnp.float32)
    m = m_ref[0, 0]
    o_ref[0] = jnp.where(m[:, None] > 0.0, y, xb)


def kernel(x, w_router, W1, W2):
    b, l, d = x.shape
    ff = W1.shape[1]
    k = max(1, int(l * _CAPACITY_RATIO))

    lblk = 256
    nblk = (b * l) // lblk

    xs = x.reshape(nblk, lblk, d)
    scores = jnp.einsum('bld,d->bl', x, w_router)

    mask = pl.pallas_call(
        functools.partial(_select_body, k=k, l=l),
        out_shape=jax.ShapeDtypeStruct((b, l), jnp.float32),
    )(scores)

    mask3 = mask.reshape(nblk, 1, lblk)
    out = pl.pallas_call(
        _mlp_body,
        grid=(nblk,),
        in_specs=[
            pl.BlockSpec((1, 1, lblk), lambda i: (i, 0, 0)),
            pl.BlockSpec((1, lblk, d), lambda i: (i, 0, 0)),
            pl.BlockSpec((d, ff), lambda i: (0, 0)),
            pl.BlockSpec((ff, d), lambda i: (0, 0)),
        ],
        out_specs=pl.BlockSpec((1, lblk, d), lambda i: (i, 0, 0)),
        out_shape=jax.ShapeDtypeStruct((nblk, lblk, d), jnp.float32),
    )(mask3, xs, W1.astype(jnp.bfloat16), W2.astype(jnp.bfloat16))

    return out.reshape(b, l, d)
